# Initial kernel scaffold; baseline (speedup 1.0000x reference)
import jax
import jax.numpy as jnp
from jax.experimental import pallas as pl


def kernel(metric, x):
    raise NotImplementedError("write your pallas kernel here")


# speedup vs baseline: 3.1223x; 3.1223x over previous
import functools

import jax
import jax.numpy as jnp
from jax import lax
from jax.experimental import pallas as pl
from jax.experimental.pallas import tpu as pltpu
from jax.experimental.pallas import tpu_sc as plsc

_R = 256
_T = 512
_C = 96
_B = 64
_TOUT = 2 * _T - _R


def _matching_body(me_ref, mo_ref, nidx_ref, rank_ref, cnt_ref, invc_ref):
    a = me_ref[0]
    b = mo_ref[0]
    scores = lax.dot_general(
        a, b, (((1,), (1,)), ((), ())),
        preferred_element_type=jnp.float32,
        precision=lax.Precision.DEFAULT,
    )
    nm_c = jnp.max(scores, axis=1, keepdims=True)
    col = lax.broadcasted_iota(jnp.int32, (_T, _T), 1)
    row = lax.broadcasted_iota(jnp.int32, (_T, _T), 0)
    nidx_c = jnp.min(jnp.where(scores == nm_c, col, _T), axis=1, keepdims=True)
    nm_r = jnp.sum(jnp.where(row == col, nm_c, 0.0), axis=0, keepdims=True)
    G = (nm_r > nm_c) | ((nm_r == nm_c) & (col < row))
    rank_c = jnp.sum(G.astype(jnp.int32), axis=1, keepdims=True)
    is_src = rank_c < _R
    K = (nidx_c == col) & is_src
    cnt_r = 1.0 + jnp.sum(K.astype(jnp.float32), axis=0, keepdims=True)
    nidx_ref[0] = nidx_c
    rank_ref[0] = rank_c
    cnt_ref[0] = cnt_r
    invc_ref[0] = 1.0 / cnt_r


def _matching(metric_even, metric_odd):
    return pl.pallas_call(
        _matching_body,
        grid=(_B,),
        in_specs=[
            pl.BlockSpec((1, _T, _C), lambda i: (i, 0, 0)),
            pl.BlockSpec((1, _T, _C), lambda i: (i, 0, 0)),
        ],
        out_specs=[
            pl.BlockSpec((1, _T, 1), lambda i: (i, 0, 0)),
            pl.BlockSpec((1, _T, 1), lambda i: (i, 0, 0)),
            pl.BlockSpec((1, 1, _T), lambda i: (i, 0, 0)),
            pl.BlockSpec((1, 1, _T), lambda i: (i, 0, 0)),
        ],
        out_shape=[
            jax.ShapeDtypeStruct((_B, _T, 1), jnp.int32),
            jax.ShapeDtypeStruct((_B, _T, 1), jnp.int32),
            jax.ShapeDtypeStruct((_B, 1, _T), jnp.float32),
            jax.ShapeDtypeStruct((_B, 1, _T), jnp.float32),
        ],
    )(metric_even, metric_odd)


def _merge_body(x_hbm, rank_hbm, nidx_hbm, cnt_hbm, invc_hbm,
                merged_hbm, sizes_hbm,
                rank_v, nidx_v, cnt_v, invc_v,
                gidx, slist, dlist, olist,
                S_v, M_v, ones_v, acc_sh):
    ci = lax.axis_index("c")
    si = lax.axis_index("s")
    wid = ci * 16 + si

    for k in range(_R // 16):
        ones_v[pl.ds(k * 16, 16)] = jnp.full((16,), 1.0, jnp.float32)

    def do_batch(t, carry):
        b = wid * 2 + t
        pltpu.sync_copy(rank_hbm.at[pl.ds(b * _T, _T)], rank_v)
        pltpu.sync_copy(nidx_hbm.at[pl.ds(b * _T, _T)], nidx_v)
        pltpu.sync_copy(cnt_hbm.at[pl.ds(b * _T, _T)], cnt_v)
        pltpu.sync_copy(invc_hbm.at[pl.ds(b * _T, _T)], invc_v)

        lane = lax.iota(jnp.int32, 16)

        def build(g, c2):
            r = rank_v[pl.ds(g * 16, 16)]
            ni = nidx_v[pl.ds(g * 16, 16)]
            i = g * 16 + lane
            xrow = b * 1024 + 2 * i
            unm = r >= _R
            p_u = jnp.where(unm, r - _R, 0)
            plsc.store_scatter(gidx, [p_u // 128, p_u % 128], xrow, mask=unm)
            p_s = jnp.where(unm, 0, r)
            plsc.store_scatter(slist, [p_s // 128, p_s % 128], xrow,
                               mask=jnp.logical_not(unm))
            plsc.store_scatter(dlist, [p_s // 128, p_s % 128], si * _T + ni,
                               mask=jnp.logical_not(unm))
            plsc.store_scatter(olist, [i // 128, i % 128], xrow + 1)
            return c2

        lax.fori_loop(0, _T // 16, build, 0)

        for j in range(_T // 128):
            pltpu.sync_copy(x_hbm.at[olist.at[j]], M_v.at[pl.ds(j * 128, 128)])
        pltpu.sync_copy(M_v, acc_sh.at[pl.ds(si * _T, _T)])
        for j in range(_R // 128):
            pltpu.sync_copy(x_hbm.at[slist.at[j]], S_v.at[pl.ds(j * 128, 128)])
        for j in range(_R // 128):
            pltpu.sync_copy(S_v.at[pl.ds(j * 128, 128)], acc_sh.at[dlist.at[j]],
                            add=True)
        for j in range(_R // 128):
            pltpu.sync_copy(x_hbm.at[gidx.at[j]], S_v.at[pl.ds(j * 128, 128)])
        pltpu.sync_copy(S_v, merged_hbm.at[pl.ds(b * _TOUT, _R)])

        pltpu.sync_copy(acc_sh.at[pl.ds(si * _T, _T)], M_v)

        def div_row(d, c2):
            inv = plsc.load_gather(invc_v, [jnp.zeros((16,), jnp.int32) + d])
            for c in range(_C // 16):
                M_v[d, pl.ds(c * 16, 16)] = M_v[d, pl.ds(c * 16, 16)] * inv
            return c2

        lax.fori_loop(0, _T, div_row, 0)

        pltpu.sync_copy(M_v, merged_hbm.at[pl.ds(b * _TOUT + _R, _T)])
        pltpu.sync_copy(ones_v, sizes_hbm.at[pl.ds(b * _TOUT, _R)])
        pltpu.sync_copy(cnt_v, sizes_hbm.at[pl.ds(b * _TOUT + _R, _T)])
        return carry

    lax.fori_loop(0, 2, do_batch, 0)


def _merge(x_flat, rank_flat, nidx_flat, cnt_flat, invc_flat):
    mesh = plsc.VectorSubcoreMesh(core_axis_name="c", subcore_axis_name="s")
    f = functools.partial(
        pl.kernel,
        mesh=mesh,
        out_type=[
            jax.ShapeDtypeStruct((_B * _TOUT, _C), jnp.float32),
            jax.ShapeDtypeStruct((_B * _TOUT,), jnp.float32),
        ],
        scratch_types=[
            pltpu.VMEM((_T,), jnp.int32),
            pltpu.VMEM((_T,), jnp.int32),
            pltpu.VMEM((_T,), jnp.float32),
            pltpu.VMEM((_T,), jnp.float32),
            pltpu.VMEM((2, 128), jnp.int32),
            pltpu.VMEM((2, 128), jnp.int32),
            pltpu.VMEM((2, 128), jnp.int32),
            pltpu.VMEM((4, 128), jnp.int32),
            pltpu.VMEM((_R, _C), jnp.float32),
            pltpu.VMEM((_T, _C), jnp.float32),
            pltpu.VMEM((_R,), jnp.float32),
            pltpu.VMEM_SHARED((16 * _T, _C), jnp.float32),
        ],
        compiler_params=pltpu.CompilerParams(
            needs_layout_passes=False, use_tc_tiling_on_sc=False),
    )(_merge_body)
    return f(x_flat, rank_flat, nidx_flat, cnt_flat, invc_flat)


def kernel(metric, x):
    m = metric / jnp.linalg.norm(metric, axis=-1, keepdims=True)
    me = m[:, ::2, :]
    mo = m[:, 1::2, :]
    nidx, rank, cnt, invc = _matching(me, mo)
    merged_flat, sizes_flat = _merge(
        x.reshape(_B * 2 * _T, _C),
        rank.reshape(_B * _T),
        nidx.reshape(_B * _T),
        cnt.reshape(_B * _T),
        invc.reshape(_B * _T),
    )
    return merged_flat.reshape(_B, _TOUT, _C), sizes_flat.reshape(_B, _TOUT)


# speedup vs baseline: 5.4150x; 1.7343x over previous
import functools

import jax
import jax.numpy as jnp
from jax import lax
from jax.experimental import pallas as pl
from jax.experimental.pallas import tpu as pltpu
from jax.experimental.pallas import tpu_sc as plsc

_R = 256
_T = 512
_C = 96
_B = 64
_TOUT = 2 * _T - _R


def _matching_body(m_ref, nidx_ref, rank_ref, cnt_ref, invc_ref):
    a = m_ref[0, :, 0, :]
    b = m_ref[0, :, 1, :]
    scores = lax.dot_general(
        a, b, (((1,), (1,)), ((), ())),
        preferred_element_type=jnp.float32,
        precision=lax.Precision.DEFAULT,
    )
    nm_c = jnp.max(scores, axis=1, keepdims=True)
    col = lax.broadcasted_iota(jnp.int32, (_T, _T), 1)
    row = lax.broadcasted_iota(jnp.int32, (_T, _T), 0)
    nidx_c = jnp.min(jnp.where(scores == nm_c, col, _T), axis=1, keepdims=True)
    nm_r = jnp.sum(jnp.where(row == col, nm_c, 0.0), axis=0, keepdims=True)
    G = (nm_r > nm_c) | ((nm_r == nm_c) & (col < row))
    rank_c = jnp.sum(G.astype(jnp.int32), axis=1, keepdims=True)
    is_src = rank_c < _R
    K = (nidx_c == col) & is_src
    cnt_r = 1.0 + jnp.sum(K.astype(jnp.float32), axis=0, keepdims=True)
    nidx_ref[0] = nidx_c
    rank_ref[0] = rank_c
    cnt_ref[0] = cnt_r
    invc_ref[0] = 1.0 / cnt_r


def _matching(m4):
    return pl.pallas_call(
        _matching_body,
        grid=(_B,),
        in_specs=[
            pl.BlockSpec((1, _T, 2, _C), lambda i: (i, 0, 0, 0)),
        ],
        out_specs=[
            pl.BlockSpec((1, _T, 1), lambda i: (i, 0, 0)),
            pl.BlockSpec((1, _T, 1), lambda i: (i, 0, 0)),
            pl.BlockSpec((1, 1, _T), lambda i: (i, 0, 0)),
            pl.BlockSpec((1, 1, _T), lambda i: (i, 0, 0)),
        ],
        out_shape=[
            jax.ShapeDtypeStruct((_B, _T, 1), jnp.int32),
            jax.ShapeDtypeStruct((_B, _T, 1), jnp.int32),
            jax.ShapeDtypeStruct((_B, 1, _T), jnp.float32),
            jax.ShapeDtypeStruct((_B, 1, _T), jnp.float32),
        ],
    )(m4)


def _merge_body(x_hbm, rank_hbm, nidx_hbm, cnt_hbm, invc_hbm,
                merged_hbm, sizes_hbm,
                rank_v, nidx_v, cnt_v, invc_v,
                gidx, slist, dlist, olist,
                S_v, M_v, ones_v, acc_sh):
    ci = lax.axis_index("c")
    si = lax.axis_index("s")
    wid = ci * 16 + si

    for k in range(_R // 16):
        ones_v[pl.ds(k * 16, 16)] = jnp.full((16,), 1.0, jnp.float32)

    def do_batch(t, carry):
        b = wid * 2 + t
        pltpu.sync_copy(rank_hbm.at[pl.ds(b * _T, _T)], rank_v)
        pltpu.sync_copy(nidx_hbm.at[pl.ds(b * _T, _T)], nidx_v)
        pltpu.sync_copy(cnt_hbm.at[pl.ds(b * _T, _T)], cnt_v)
        pltpu.sync_copy(invc_hbm.at[pl.ds(b * _T, _T)], invc_v)

        lane = lax.iota(jnp.int32, 16)

        def build(g, c2):
            r = rank_v[pl.ds(g * 16, 16)]
            ni = nidx_v[pl.ds(g * 16, 16)]
            i = g * 16 + lane
            xrow = b * 1024 + 2 * i
            unm = r >= _R
            p_u = jnp.where(unm, r - _R, 0)
            plsc.store_scatter(gidx, [p_u // 128, p_u % 128], xrow, mask=unm)
            p_s = jnp.where(unm, 0, r)
            plsc.store_scatter(slist, [p_s // 128, p_s % 128], xrow,
                               mask=jnp.logical_not(unm))
            plsc.store_scatter(dlist, [p_s // 128, p_s % 128], si * _T + ni,
                               mask=jnp.logical_not(unm))
            plsc.store_scatter(olist, [i // 128, i % 128], xrow + 1)
            return c2

        lax.fori_loop(0, _T // 16, build, 0)

        for j in range(_T // 128):
            pltpu.sync_copy(x_hbm.at[olist.at[j]], M_v.at[pl.ds(j * 128, 128)])
        pltpu.sync_copy(M_v, acc_sh.at[pl.ds(si * _T, _T)])
        for j in range(_R // 128):
            pltpu.sync_copy(x_hbm.at[slist.at[j]], S_v.at[pl.ds(j * 128, 128)])
        for j in range(_R // 128):
            pltpu.sync_copy(S_v.at[pl.ds(j * 128, 128)], acc_sh.at[dlist.at[j]],
                            add=True)
        for j in range(_R // 128):
            pltpu.sync_copy(x_hbm.at[gidx.at[j]], S_v.at[pl.ds(j * 128, 128)])
        pltpu.sync_copy(S_v, merged_hbm.at[pl.ds(b * _TOUT, _R)])

        pltpu.sync_copy(acc_sh.at[pl.ds(si * _T, _T)], M_v)

        def div_row(d, c2):
            inv = plsc.load_gather(invc_v, [jnp.zeros((16,), jnp.int32) + d])
            for c in range(_C // 16):
                M_v[d, pl.ds(c * 16, 16)] = M_v[d, pl.ds(c * 16, 16)] * inv
            return c2

        lax.fori_loop(0, _T, div_row, 0)

        pltpu.sync_copy(M_v, merged_hbm.at[pl.ds(b * _TOUT + _R, _T)])
        pltpu.sync_copy(ones_v, sizes_hbm.at[pl.ds(b * _TOUT, _R)])
        pltpu.sync_copy(cnt_v, sizes_hbm.at[pl.ds(b * _TOUT + _R, _T)])
        return carry

    lax.fori_loop(0, 2, do_batch, 0)


def _merge(x_flat, rank_flat, nidx_flat, cnt_flat, invc_flat):
    mesh = plsc.VectorSubcoreMesh(core_axis_name="c", subcore_axis_name="s")
    f = functools.partial(
        pl.kernel,
        mesh=mesh,
        out_type=[
            jax.ShapeDtypeStruct((_B * _TOUT, _C), jnp.float32),
            jax.ShapeDtypeStruct((_B * _TOUT,), jnp.float32),
        ],
        scratch_types=[
            pltpu.VMEM((_T,), jnp.int32),
            pltpu.VMEM((_T,), jnp.int32),
            pltpu.VMEM((_T,), jnp.float32),
            pltpu.VMEM((_T,), jnp.float32),
            pltpu.VMEM((2, 128), jnp.int32),
            pltpu.VMEM((2, 128), jnp.int32),
            pltpu.VMEM((2, 128), jnp.int32),
            pltpu.VMEM((4, 128), jnp.int32),
            pltpu.VMEM((_R, _C), jnp.float32),
            pltpu.VMEM((_T, _C), jnp.float32),
            pltpu.VMEM((_R,), jnp.float32),
            pltpu.VMEM_SHARED((16 * _T, _C), jnp.float32),
        ],
        compiler_params=pltpu.CompilerParams(
            needs_layout_passes=False, use_tc_tiling_on_sc=False),
    )(_merge_body)
    return f(x_flat, rank_flat, nidx_flat, cnt_flat, invc_flat)


def kernel(metric, x):
    m = metric / jnp.linalg.norm(metric, axis=-1, keepdims=True)
    nidx, rank, cnt, invc = _matching(m.reshape(_B, _T, 2, _C))
    merged_flat, sizes_flat = _merge(
        x.reshape(_B * 2 * _T, _C),
        rank.reshape(_B * _T),
        nidx.reshape(_B * _T),
        cnt.reshape(_B * _T),
        invc.reshape(_B * _T),
    )
    return merged_flat.reshape(_B, _TOUT, _C), sizes_flat.reshape(_B, _TOUT)
